# baseline (speedup 1.0000x reference)
import functools

import jax
import jax.numpy as jnp
from jax import lax
from jax.experimental import pallas as pl
from jax.experimental.pallas import tpu as pltpu
from jax.experimental.pallas import tpu_sc as plsc

_EMBED = 64
_PAD = 128
_BATCH = 4096
_N_NEG = 50
_NW = 32
_BPW = _BATCH // _NW
_CHUNK = 128


def _body(users_hbm, pos_hbm, pot_hbm, negt_hbm, uemb_hbm, iemb_hbm,
          out_u, out_p, out_t, out_n,
          sidx_v, negidx_v, blk_v, wblk_v, gsems, wsems):
    wid = lax.axis_index("s") * 2 + lax.axis_index("c")
    base = wid * _BPW
    smalls = ((uemb_hbm, out_u), (iemb_hbm, out_p), (iemb_hbm, out_t))

    for i, src in enumerate((users_hbm, pos_hbm, pot_hbm)):
        pltpu.sync_copy(src.at[pl.ds(base, _BPW)], sidx_v.at[i])
    pltpu.sync_copy(negt_hbm.at[:, pl.ds(base, _BPW)], negidx_v)

    rows8 = [jnp.int32(l0 * 16) + lax.iota(jnp.int32, 16) for l0 in range(8)]

    def transpose_blk(b):
        def erow(e, c):
            cols = lax.broadcast(e, (16,))
            for l0 in range(8):
                v = plsc.load_gather(blk_v.at[b], [rows8[l0], cols])
                wblk_v[b, e, pl.ds(l0 * 16, 16)] = v
            return c
        lax.fori_loop(0, _EMBED, erow, 0)

    def fire_gather_small(i, b):
        table = smalls[i][0]
        pltpu.async_copy(table.at[sidx_v.at[i]], blk_v.at[b], gsems[b])

    def fire_gather_neg(j, b):
        pltpu.async_copy(iemb_hbm.at[negidx_v.at[j]], blk_v.at[b], gsems[b])

    def wait_gather(b):
        pltpu.make_async_copy(iemb_hbm.at[pl.ds(0, _CHUNK)], blk_v.at[b],
                              gsems[b]).wait()

    def fire_wb_small(i, b):
        pltpu.async_copy(wblk_v.at[b], smalls[i][1].at[:, pl.ds(base, _BPW)],
                         wsems[b])

    def fire_wb_neg(j, b):
        pltpu.async_copy(wblk_v.at[b], out_n.at[j, :, pl.ds(base, _BPW)],
                         wsems[b])

    def wait_wb(b):
        pltpu.make_async_copy(wblk_v.at[b], out_n.at[0, :, pl.ds(base, _BPW)],
                              wsems[b]).wait()

    fire_gather_small(0, 0)
    fire_gather_small(1, 1)
    wait_gather(0)
    transpose_blk(0)
    fire_wb_small(0, 0)
    fire_gather_small(2, 0)
    wait_gather(1)
    transpose_blk(1)
    fire_wb_small(1, 1)
    fire_gather_neg(0, 1)
    wait_gather(0)
    wait_wb(0)
    transpose_blk(0)
    fire_wb_small(2, 0)
    fire_gather_neg(1, 0)

    def step(j, par):
        wait_gather(par)
        wait_wb(par)
        transpose_blk(par)
        fire_wb_neg(j, par)
        fire_gather_neg(j + 2, par)

    def super_step(k, c):
        step(2 * k, 1)
        step(2 * k + 1, 0)
        return c

    lax.fori_loop(0, (_N_NEG - 2) // 2, super_step, 0)

    for j, par in ((_N_NEG - 2, 1), (_N_NEG - 1, 0)):
        wait_gather(par)
        wait_wb(par)
        transpose_blk(par)
        fire_wb_neg(j, par)
    wait_wb(1)
    wait_wb(0)


@functools.partial(
    pl.kernel,
    mesh=plsc.VectorSubcoreMesh(core_axis_name="c", subcore_axis_name="s"),
    out_type=(
        jax.ShapeDtypeStruct((_EMBED, _BATCH), jnp.float32),
        jax.ShapeDtypeStruct((_EMBED, _BATCH), jnp.float32),
        jax.ShapeDtypeStruct((_EMBED, _BATCH), jnp.float32),
        jax.ShapeDtypeStruct((_N_NEG, _EMBED, _BATCH), jnp.float32),
    ),
    scratch_types=[
        pltpu.VMEM((3, _BPW), jnp.int32),
        pltpu.VMEM((_N_NEG, _BPW), jnp.int32),
        pltpu.VMEM((2, _CHUNK, _PAD), jnp.float32),
        pltpu.VMEM((2, _EMBED, _BPW), jnp.float32),
        [pltpu.SemaphoreType.DMA] * 2,
        [pltpu.SemaphoreType.DMA] * 2,
    ],
    compiler_params=pltpu.CompilerParams(needs_layout_passes=False),
)
def _sc_gather(*refs):
    _body(*refs)


def kernel(users, pos_items, pot_items, neg_items, user_emb, item_emb):
    uemb_p = jnp.pad(user_emb, ((0, 0), (0, _PAD - _EMBED)))
    iemb_p = jnp.pad(item_emb, ((0, 0), (0, _PAD - _EMBED)))
    out_ut, out_pt, out_tt, out_nt = _sc_gather(
        users.astype(jnp.int32), pos_items, pot_items, neg_items.T,
        uemb_p, iemb_p)
    return (out_ut.T, out_pt.T, out_tt.T, out_nt.transpose(2, 0, 1))
